# baseline (device time: 722913 ns/iter reference)
import jax
import jax.numpy as jnp
from jax import lax
from jax.experimental import pallas as pl
from jax.experimental.pallas import tpu as pltpu

N_DEV = 16


def kernel(x, w_mat):
    m_per, k = x.shape
    _, n_per = w_mat.shape

    def body(x_ref, w_ref, out_ref, comm_ref, send_sems, recv_sems,
             amax_src, amax_buf, amax_send_sems, amax_recv_sems):
        my = lax.axis_index("i")
        left = lax.rem(my + N_DEV - 1, N_DEV)
        right = lax.rem(my + 1, N_DEV)

        amax_buf[...] = jnp.zeros_like(amax_buf)

        barrier_sem = pltpu.get_barrier_semaphore()
        for nbr in (left, right):
            pl.semaphore_signal(barrier_sem, inc=1, device_id=(nbr,),
                                device_id_type=pl.DeviceIdType.MESH)
        pl.semaphore_wait(barrier_sem, 2)

        def gemm(chunk, origin):
            out_ref[pl.ds(origin * m_per, m_per), :] = jnp.dot(
                chunk, w_ref[...], preferred_element_type=jnp.float32)

        for h in range(N_DEV - 1):
            src = x_ref if h == 0 else comm_ref.at[(h - 1) % 2]
            rdma = pltpu.make_async_remote_copy(
                src_ref=src,
                dst_ref=comm_ref.at[h % 2],
                send_sem=send_sems.at[h % 2],
                recv_sem=recv_sems.at[h % 2],
                device_id=(right,),
                device_id_type=pl.DeviceIdType.MESH,
            )
            rdma.start()
            if h == 0:
                gemm(x_ref[...], my)
            else:
                gemm(comm_ref[(h - 1) % 2], lax.rem(my - h + N_DEV, N_DEV))
            rdma.wait()
        gemm(comm_ref[(N_DEV - 2) % 2], right)

        local_amax = jnp.max(jnp.abs(out_ref[...]))
        amax_src[...] = jnp.broadcast_to(local_amax, amax_src.shape)

        rdmas = []
        for j in range(1, N_DEV):
            rdma = pltpu.make_async_remote_copy(
                src_ref=amax_src,
                dst_ref=amax_buf.at[pl.ds(my, 1)],
                send_sem=amax_send_sems.at[j - 1],
                recv_sem=amax_recv_sems.at[j - 1],
                device_id=(lax.rem(my + j, N_DEV),),
                device_id_type=pl.DeviceIdType.MESH,
            )
            rdma.start()
            rdmas.append(rdma)
        for rdma in rdmas:
            rdma.wait()

        gmax = jnp.maximum(jnp.max(amax_buf[...]), local_amax)
        scale = gmax / 448.0
        y = out_ref[...]
        q = jnp.clip(y / scale, -448.0, 448.0).astype(jnp.float8_e4m3fn)
        out_ref[...] = q.astype(jnp.float32) * scale

    return pl.pallas_call(
        body,
        out_shape=jax.ShapeDtypeStruct((N_DEV * m_per, n_per), jnp.float32),
        in_specs=[pl.BlockSpec(memory_space=pltpu.VMEM),
                  pl.BlockSpec(memory_space=pltpu.VMEM)],
        out_specs=pl.BlockSpec(memory_space=pltpu.VMEM),
        scratch_shapes=[
            pltpu.VMEM((2, m_per, k), jnp.float32),
            pltpu.SemaphoreType.DMA((2,)),
            pltpu.SemaphoreType.DMA((2,)),
            pltpu.VMEM((1, 128), jnp.float32),
            pltpu.VMEM((N_DEV, 128), jnp.float32),
            pltpu.SemaphoreType.DMA((N_DEV - 1,)),
            pltpu.SemaphoreType.DMA((N_DEV - 1,)),
        ],
        compiler_params=pltpu.CompilerParams(collective_id=0),
    )(x, w_mat)


# device time: 404751 ns/iter; 1.7861x vs baseline; 1.7861x over previous
import jax
import jax.numpy as jnp
from jax import lax
from jax.experimental import pallas as pl
from jax.experimental.pallas import tpu as pltpu

N_DEV = 16


def kernel(x, w_mat):
    m_per, k = x.shape
    _, n_per = w_mat.shape
    m_half = m_per // 2

    def body(x_ref, w_ref, out_ref, comm_r, comm_l, sems_s, sems_r,
             amax_src, amax_buf, amax_send_sems, amax_recv_sems):
        my = lax.axis_index("i")
        left = lax.rem(my + N_DEV - 1, N_DEV)
        right = lax.rem(my + 1, N_DEV)

        amax_buf[...] = jnp.zeros_like(amax_buf)

        barrier_sem = pltpu.get_barrier_semaphore()
        for nbr in (left, right):
            pl.semaphore_signal(barrier_sem, inc=1, device_id=(nbr,),
                                device_id_type=pl.DeviceIdType.MESH)
        pl.semaphore_wait(barrier_sem, 2)

        def gemm_half(chunk, origin, half):
            out_ref[pl.ds(origin * m_per + half * m_half, m_half), :] = (
                jnp.dot(chunk, w_ref[...], preferred_element_type=jnp.float32))

        for h in range(N_DEV - 1):
            src_r = x_ref.at[pl.ds(0, m_half)] if h == 0 else comm_r.at[(h - 1) % 2]
            src_l = x_ref.at[pl.ds(m_half, m_half)] if h == 0 else comm_l.at[(h - 1) % 2]
            rdma_r = pltpu.make_async_remote_copy(
                src_ref=src_r,
                dst_ref=comm_r.at[h % 2],
                send_sem=sems_s.at[0, h % 2],
                recv_sem=sems_r.at[0, h % 2],
                device_id=(right,),
                device_id_type=pl.DeviceIdType.MESH,
            )
            rdma_l = pltpu.make_async_remote_copy(
                src_ref=src_l,
                dst_ref=comm_l.at[h % 2],
                send_sem=sems_s.at[1, h % 2],
                recv_sem=sems_r.at[1, h % 2],
                device_id=(left,),
                device_id_type=pl.DeviceIdType.MESH,
            )
            rdma_r.start()
            rdma_l.start()
            if h == 0:
                gemm_half(x_ref[pl.ds(0, m_half), :], my, 0)
                gemm_half(x_ref[pl.ds(m_half, m_half), :], my, 1)
            else:
                gemm_half(comm_r[(h - 1) % 2], lax.rem(my - h + N_DEV, N_DEV), 0)
                gemm_half(comm_l[(h - 1) % 2], lax.rem(my + h, N_DEV), 1)
            rdma_r.wait()
            rdma_l.wait()
        last = (N_DEV - 2) % 2
        gemm_half(comm_r[last], right, 0)
        gemm_half(comm_l[last], left, 1)

        local_amax = jnp.max(jnp.abs(out_ref[...]))
        amax_src[...] = jnp.broadcast_to(local_amax, amax_src.shape)

        rdmas = []
        for j in range(1, N_DEV):
            rdma = pltpu.make_async_remote_copy(
                src_ref=amax_src,
                dst_ref=amax_buf.at[pl.ds(my, 1)],
                send_sem=amax_send_sems.at[j - 1],
                recv_sem=amax_recv_sems.at[j - 1],
                device_id=(lax.rem(my + j, N_DEV),),
                device_id_type=pl.DeviceIdType.MESH,
            )
            rdma.start()
            rdmas.append(rdma)
        for rdma in rdmas:
            rdma.wait()

        gmax = jnp.maximum(jnp.max(amax_buf[...]), local_amax)
        scale = gmax / 448.0
        y = out_ref[...]
        q = jnp.clip(y / scale, -448.0, 448.0).astype(jnp.float8_e4m3fn)
        out_ref[...] = q.astype(jnp.float32) * scale

    return pl.pallas_call(
        body,
        out_shape=jax.ShapeDtypeStruct((N_DEV * m_per, n_per), jnp.float32),
        in_specs=[pl.BlockSpec(memory_space=pltpu.VMEM),
                  pl.BlockSpec(memory_space=pltpu.VMEM)],
        out_specs=pl.BlockSpec(memory_space=pltpu.VMEM),
        scratch_shapes=[
            pltpu.VMEM((2, m_half, k), jnp.float32),
            pltpu.VMEM((2, m_half, k), jnp.float32),
            pltpu.SemaphoreType.DMA((2, 2)),
            pltpu.SemaphoreType.DMA((2, 2)),
            pltpu.VMEM((1, 128), jnp.float32),
            pltpu.VMEM((N_DEV, 128), jnp.float32),
            pltpu.SemaphoreType.DMA((N_DEV - 1,)),
            pltpu.SemaphoreType.DMA((N_DEV - 1,)),
        ],
        compiler_params=pltpu.CompilerParams(collective_id=0),
    )(x, w_mat)
